# baseline (speedup 1.0000x reference)
import functools

import jax
import jax.numpy as jnp
from jax import lax
from jax.experimental import pallas as pl
from jax.experimental.pallas import tpu as pltpu
from jax.experimental.pallas import tpu_sc as plsc

_G = 8


def _mesh():
    return plsc.VectorSubcoreMesh(core_axis_name="c", subcore_axis_name="s")


@jax.jit
def _sc_repack(table_t):
    d, v = table_t.shape
    n_full = v // 128
    n_tail = v - n_full * 128
    assert d % 16 == 0 and 128 % d == 0

    @functools.partial(
        pl.kernel,
        out_type=jax.ShapeDtypeStruct((v, d), jnp.float32),
        mesh=_mesh(),
        scratch_types=[
            pltpu.VMEM((d, 128), jnp.float32),
            pltpu.VMEM((128, d), jnp.float32),
        ],
        compiler_params=pltpu.CompilerParams(use_tc_tiling_on_sc=False,
                                             needs_layout_passes=False),
    )
    def k(tt_hbm, out_hbm, ibuf, obuf):
        nc = lax.axis_size("c")
        nw = nc * lax.axis_size("s")
        wid = lax.axis_index("s") * nc + lax.axis_index("c")

        lane = lax.iota(jnp.int32, 16)
        evecs = [lane + 16 * m for m in range(d // 16)]

        def transpose_rows(rows):
            for r in range(rows):
                for m, ev in enumerate(evecs):
                    vals = plsc.load_gather(
                        ibuf, [ev, jnp.full((16,), r, jnp.int32)])
                    obuf[r, pl.ds(16 * m, 16)] = vals

        def body(i, carry):
            c = wid + i * nw

            @pl.when(c < n_full)
            def _():
                pltpu.sync_copy(tt_hbm.at[:, pl.ds(c * 128, 128)], ibuf)
                transpose_rows(128)
                pltpu.sync_copy(obuf, out_hbm.at[pl.ds(c * 128, 128), :])

            if n_tail:
                @pl.when(c == n_full)
                def _():
                    pltpu.sync_copy(tt_hbm.at[:, pl.ds(n_full * 128, n_tail)],
                                    ibuf.at[:, pl.ds(0, n_tail)])
                    transpose_rows(n_tail)
                    pltpu.sync_copy(
                        obuf.at[pl.ds(0, n_tail)],
                        out_hbm.at[pl.ds(n_full * 128, n_tail), :])
            return carry

        n_units = n_full + (1 if n_tail else 0)
        lax.fori_loop(0, (n_units - wid + nw - 1) // nw, body, 0)

    return k(table_t)


@functools.partial(jax.jit, static_argnums=(2,))
def _sc_gather(idx, table, n_workers):
    b, h = idx.shape
    _, d = table.shape
    b_per_w = b // n_workers
    groups = b_per_w // _G
    assert groups % 2 == 0 and groups >= 4

    @functools.partial(
        pl.kernel,
        out_type=jax.ShapeDtypeStruct((b, h, d), jnp.float32),
        mesh=_mesh(),
        scratch_types=[
            pltpu.VMEM((_G, h), jnp.int32),
            pltpu.VMEM((_G, h), jnp.int32),
            pltpu.VMEM((_G, h, d), jnp.float32),
            pltpu.VMEM((_G, h, d), jnp.float32),
            pltpu.SemaphoreType.DMA,
            pltpu.SemaphoreType.DMA,
            pltpu.SemaphoreType.DMA,
            pltpu.SemaphoreType.DMA,
        ],
        compiler_params=pltpu.CompilerParams(use_tc_tiling_on_sc=False),
    )
    def k(idx_hbm, table_hbm, out_hbm, idx0, idx1, rows0, rows1,
          gsem0, gsem1, ssem0, ssem1):
        idxv = (idx0, idx1)
        rows = (rows0, rows1)
        gsem = (gsem0, gsem1)
        ssem = (ssem0, ssem1)

        nc = lax.axis_size("c")
        wid = lax.axis_index("s") * nc + lax.axis_index("c")
        b0 = wid * b_per_w

        def fire_gathers(g, buf):
            pltpu.sync_copy(idx_hbm.at[pl.ds(b0 + g * _G, _G)], idxv[buf])
            for j in range(_G):
                pltpu.async_copy(
                    table_hbm.at[idxv[buf].at[j]],
                    rows[buf].at[j],
                    gsem[buf],
                )

        def drain_gathers(buf):
            for j in range(_G):
                pltpu.make_async_copy(
                    table_hbm.at[idxv[buf].at[j]],
                    rows[buf].at[j],
                    gsem[buf],
                ).wait()

        def fire_store(g, buf):
            pltpu.async_copy(
                rows[buf], out_hbm.at[pl.ds(b0 + g * _G, _G)], ssem[buf])

        def drain_store(g, buf):
            pltpu.make_async_copy(
                rows[buf], out_hbm.at[pl.ds(b0 + g * _G, _G)], ssem[buf]
            ).wait()

        fire_gathers(0, 0)
        fire_gathers(1, 1)
        drain_gathers(0)
        fire_store(0, 0)

        def body(j, carry):
            g = 2 * j + 1
            drain_store(g - 1, 0)
            fire_gathers(g + 1, 0)
            drain_gathers(1)
            fire_store(g, 1)

            drain_store(g, 1)
            fire_gathers(g + 2, 1)
            drain_gathers(0)
            fire_store(g + 1, 0)
            return carry

        lax.fori_loop(0, (groups - 2) // 2, body, 0)

        drain_gathers(1)
        fire_store(groups - 1, 1)
        drain_store(groups - 2, 0)
        drain_store(groups - 1, 1)

    return k(idx, table)


def kernel(input, embedding_matrix):
    b, h = input.shape

    info = plsc.get_sparse_core_info()
    n_workers = info.num_cores * info.num_subcores

    assert b % (n_workers * _G) == 0
    table_t = jnp.swapaxes(embedding_matrix, 0, 1)
    table_l = _sc_repack(table_t)
    return _sc_gather(input.astype(jnp.int32), table_l, n_workers)


# speedup vs baseline: 2.9969x; 2.9969x over previous
import functools

import jax
import jax.numpy as jnp
from jax import lax
from jax.experimental import pallas as pl
from jax.experimental.pallas import tpu as pltpu
from jax.experimental.pallas import tpu_sc as plsc

_G = 8


def _mesh():
    return plsc.VectorSubcoreMesh(core_axis_name="c", subcore_axis_name="s")


_CHUNK = 256


@jax.jit
def _sc_repack(table):
    v, d = table.shape
    n_full = v // _CHUNK
    n_tail = v - n_full * _CHUNK
    rows_c = _CHUNK * d // 128
    assert d % 32 == 0 and 128 % d == 0 and n_tail % (128 // d) == 0

    @functools.partial(
        pl.kernel,
        out_type=jax.ShapeDtypeStruct((v * d // 128, 128), jnp.float32),
        mesh=_mesh(),
        scratch_types=[
            pltpu.VMEM((_CHUNK, d), jnp.float32),
            pltpu.VMEM((rows_c, 128), jnp.float32),
        ],
        compiler_params=pltpu.CompilerParams(use_tc_tiling_on_sc=True,
                                             needs_layout_passes=False),
    )
    def k(tab_hbm, out_hbm, ibuf, obuf):
        nc = lax.axis_size("c")
        nw = nc * lax.axis_size("s")
        wid = lax.axis_index("s") * nc + lax.axis_index("c")
        rpo = 128 // d

        def repack_rows(n_out_rows):
            def q_body(q, carry):
                for m in range(8):
                    obuf[q, pl.ds(16 * m, 16)] = (
                        ibuf[q * rpo + (16 * m) // d,
                             pl.ds((16 * m) % d, 16)])
                return carry
            lax.fori_loop(0, n_out_rows, q_body, 0)

        def body(i, carry):
            c = wid + i * nw

            @pl.when(c < n_full)
            def _():
                pltpu.sync_copy(tab_hbm.at[pl.ds(c * _CHUNK, _CHUNK), :], ibuf)
                repack_rows(rows_c)
                pltpu.sync_copy(obuf, out_hbm.at[pl.ds(c * rows_c, rows_c)])

            if n_tail:
                @pl.when(c == n_full)
                def _():
                    tr = n_tail * d // 128
                    pltpu.sync_copy(tab_hbm.at[pl.ds(n_full * _CHUNK, n_tail), :],
                                    ibuf.at[pl.ds(0, n_tail)])
                    repack_rows(tr)
                    pltpu.sync_copy(obuf.at[pl.ds(0, tr)],
                                    out_hbm.at[pl.ds(n_full * rows_c, tr)])
            return carry

        n_units = n_full + (1 if n_tail else 0)
        lax.fori_loop(0, (n_units - wid + nw - 1) // nw, body, 0)

    return k(table)


@functools.partial(jax.jit, static_argnums=(2,))
def _sc_gather(idx, table, n_workers):
    b, h = idx.shape
    _, d = table.shape
    b_per_w = b // n_workers
    groups = b_per_w // _G
    assert groups % 2 == 0 and groups >= 4

    @functools.partial(
        pl.kernel,
        out_type=jax.ShapeDtypeStruct((b, h, d), jnp.float32),
        mesh=_mesh(),
        scratch_types=[
            pltpu.VMEM((_G, h), jnp.int32),
            pltpu.VMEM((_G, h), jnp.int32),
            pltpu.VMEM((_G, h, d), jnp.float32),
            pltpu.VMEM((_G, h, d), jnp.float32),
            pltpu.SemaphoreType.DMA,
            pltpu.SemaphoreType.DMA,
            pltpu.SemaphoreType.DMA,
            pltpu.SemaphoreType.DMA,
        ],
        compiler_params=pltpu.CompilerParams(use_tc_tiling_on_sc=False),
    )
    def k(idx_hbm, table_hbm, out_hbm, idx0, idx1, rows0, rows1,
          gsem0, gsem1, ssem0, ssem1):
        idxv = (idx0, idx1)
        rows = (rows0, rows1)
        gsem = (gsem0, gsem1)
        ssem = (ssem0, ssem1)

        nc = lax.axis_size("c")
        wid = lax.axis_index("s") * nc + lax.axis_index("c")
        b0 = wid * b_per_w

        def fire_gathers(g, buf):
            pltpu.sync_copy(idx_hbm.at[pl.ds(b0 + g * _G, _G)], idxv[buf])
            for j in range(_G):
                pltpu.async_copy(
                    table_hbm.at[idxv[buf].at[j]],
                    rows[buf].at[j],
                    gsem[buf],
                )

        def drain_gathers(buf):
            for j in range(_G):
                pltpu.make_async_copy(
                    table_hbm.at[idxv[buf].at[j]],
                    rows[buf].at[j],
                    gsem[buf],
                ).wait()

        def fire_store(g, buf):
            pltpu.async_copy(
                rows[buf], out_hbm.at[pl.ds(b0 + g * _G, _G)], ssem[buf])

        def drain_store(g, buf):
            pltpu.make_async_copy(
                rows[buf], out_hbm.at[pl.ds(b0 + g * _G, _G)], ssem[buf]
            ).wait()

        fire_gathers(0, 0)
        fire_gathers(1, 1)
        drain_gathers(0)
        fire_store(0, 0)

        def body(j, carry):
            g = 2 * j + 1
            drain_store(g - 1, 0)
            fire_gathers(g + 1, 0)
            drain_gathers(1)
            fire_store(g, 1)

            drain_store(g, 1)
            fire_gathers(g + 2, 1)
            drain_gathers(0)
            fire_store(g + 1, 0)
            return carry

        lax.fori_loop(0, (groups - 2) // 2, body, 0)

        drain_gathers(1)
        fire_store(groups - 1, 1)
        drain_store(groups - 2, 0)
        drain_store(groups - 1, 1)

    return k(idx, table)


def kernel(input, embedding_matrix):
    b, h = input.shape

    info = plsc.get_sparse_core_info()
    n_workers = info.num_cores * info.num_subcores

    assert b % (n_workers * _G) == 0
    v, d = embedding_matrix.shape
    table_l = _sc_repack(embedding_matrix).reshape(v, d)
    return _sc_gather(input.astype(jnp.int32), table_l, n_workers)


# speedup vs baseline: 3.8840x; 1.2960x over previous
import functools

import jax
import jax.numpy as jnp
from jax import lax
from jax.experimental import pallas as pl
from jax.experimental.pallas import tpu as pltpu
from jax.experimental.pallas import tpu_sc as plsc

_G = 8


@functools.partial(jax.jit, static_argnums=(2,))
def _sc_gather(idx, table, n_workers):
    b, h = idx.shape
    _, d = table.shape
    b_per_w = b // n_workers
    groups = b_per_w // _G
    assert groups % 2 == 0 and groups >= 4

    mesh = plsc.VectorSubcoreMesh(core_axis_name="c", subcore_axis_name="s")

    @functools.partial(
        pl.kernel,
        out_type=jax.ShapeDtypeStruct((b, h, d), jnp.float32),
        mesh=mesh,
        scratch_types=[
            pltpu.VMEM((_G, h), jnp.int32),
            pltpu.VMEM((_G, h), jnp.int32),
            pltpu.VMEM((_G, h, d), jnp.float32),
            pltpu.VMEM((_G, h, d), jnp.float32),
            pltpu.SemaphoreType.DMA,
            pltpu.SemaphoreType.DMA,
            pltpu.SemaphoreType.DMA,
            pltpu.SemaphoreType.DMA,
        ],
        compiler_params=pltpu.CompilerParams(use_tc_tiling_on_sc=False),
    )
    def k(idx_hbm, table_hbm, out_hbm, idx0, idx1, rows0, rows1,
          gsem0, gsem1, ssem0, ssem1):
        idxv = (idx0, idx1)
        rows = (rows0, rows1)
        gsem = (gsem0, gsem1)
        ssem = (ssem0, ssem1)

        nc = lax.axis_size("c")
        wid = lax.axis_index("s") * nc + lax.axis_index("c")
        b0 = wid * b_per_w

        def fire_gathers(g, buf):
            pltpu.sync_copy(idx_hbm.at[pl.ds(b0 + g * _G, _G)], idxv[buf])
            for j in range(_G):
                pltpu.async_copy(
                    table_hbm.at[idxv[buf].at[j]],
                    rows[buf].at[j],
                    gsem[buf],
                )

        def drain_gathers(buf):
            for j in range(_G):
                pltpu.make_async_copy(
                    table_hbm.at[idxv[buf].at[j]],
                    rows[buf].at[j],
                    gsem[buf],
                ).wait()

        def fire_store(g, buf):
            pltpu.async_copy(
                rows[buf], out_hbm.at[pl.ds(b0 + g * _G, _G)], ssem[buf])

        def drain_store(g, buf):
            pltpu.make_async_copy(
                rows[buf], out_hbm.at[pl.ds(b0 + g * _G, _G)], ssem[buf]
            ).wait()

        fire_gathers(0, 0)
        fire_gathers(1, 1)
        drain_gathers(0)
        fire_store(0, 0)

        def body(j, carry):
            g = 2 * j + 1
            drain_store(g - 1, 0)
            fire_gathers(g + 1, 0)
            drain_gathers(1)
            fire_store(g, 1)

            drain_store(g, 1)
            fire_gathers(g + 2, 1)
            drain_gathers(0)
            fire_store(g + 1, 0)
            return carry

        lax.fori_loop(0, (groups - 2) // 2, body, 0)

        drain_gathers(1)
        fire_store(groups - 1, 1)
        drain_store(groups - 2, 0)
        drain_store(groups - 1, 1)

    return k(idx, table)


def kernel(input, embedding_matrix):
    b, h = input.shape

    info = plsc.get_sparse_core_info()
    n_workers = info.num_cores * info.num_subcores

    assert b % (n_workers * _G) == 0
    return _sc_gather(input.astype(jnp.int32), embedding_matrix, n_workers)


# speedup vs baseline: 3.9321x; 1.0124x over previous
import functools

import jax
import jax.numpy as jnp
from jax import lax
from jax.experimental import pallas as pl
from jax.experimental.pallas import tpu as pltpu
from jax.experimental.pallas import tpu_sc as plsc

_G = 16


@functools.partial(jax.jit, static_argnums=(2,))
def _sc_gather(idx, table, n_workers):
    b, h = idx.shape
    _, d = table.shape
    b_per_w = b // n_workers
    groups = b_per_w // _G
    assert groups % 2 == 0 and groups >= 4

    mesh = plsc.VectorSubcoreMesh(core_axis_name="c", subcore_axis_name="s")

    @functools.partial(
        pl.kernel,
        out_type=jax.ShapeDtypeStruct((b, h, d), jnp.float32),
        mesh=mesh,
        scratch_types=[
            pltpu.VMEM((_G, h), jnp.int32),
            pltpu.VMEM((_G, h), jnp.int32),
            pltpu.VMEM((_G, h, d), jnp.float32),
            pltpu.VMEM((_G, h, d), jnp.float32),
            pltpu.SemaphoreType.DMA,
            pltpu.SemaphoreType.DMA,
            pltpu.SemaphoreType.DMA,
            pltpu.SemaphoreType.DMA,
        ],
        compiler_params=pltpu.CompilerParams(use_tc_tiling_on_sc=False),
    )
    def k(idx_hbm, table_hbm, out_hbm, idx0, idx1, rows0, rows1,
          gsem0, gsem1, ssem0, ssem1):
        idxv = (idx0, idx1)
        rows = (rows0, rows1)
        gsem = (gsem0, gsem1)
        ssem = (ssem0, ssem1)

        nc = lax.axis_size("c")
        wid = lax.axis_index("s") * nc + lax.axis_index("c")
        b0 = wid * b_per_w

        def fire_gathers(g, buf):
            pltpu.sync_copy(idx_hbm.at[pl.ds(b0 + g * _G, _G)], idxv[buf])
            for j in range(_G):
                pltpu.async_copy(
                    table_hbm.at[idxv[buf].at[j]],
                    rows[buf].at[j],
                    gsem[buf],
                )

        def drain_gathers(buf):
            for j in range(_G):
                pltpu.make_async_copy(
                    table_hbm.at[idxv[buf].at[j]],
                    rows[buf].at[j],
                    gsem[buf],
                ).wait()

        def fire_store(g, buf):
            pltpu.async_copy(
                rows[buf], out_hbm.at[pl.ds(b0 + g * _G, _G)], ssem[buf])

        def drain_store(g, buf):
            pltpu.make_async_copy(
                rows[buf], out_hbm.at[pl.ds(b0 + g * _G, _G)], ssem[buf]
            ).wait()

        fire_gathers(0, 0)
        fire_gathers(1, 1)
        drain_gathers(0)
        fire_store(0, 0)

        def body(j, carry):
            g = 2 * j + 1
            drain_store(g - 1, 0)
            fire_gathers(g + 1, 0)
            drain_gathers(1)
            fire_store(g, 1)

            drain_store(g, 1)
            fire_gathers(g + 2, 1)
            drain_gathers(0)
            fire_store(g + 1, 0)
            return carry

        lax.fori_loop(0, (groups - 2) // 2, body, 0)

        drain_gathers(1)
        fire_store(groups - 1, 1)
        drain_store(groups - 2, 0)
        drain_store(groups - 1, 1)

    return k(idx, table)


def kernel(input, embedding_matrix):
    b, h = input.shape

    info = plsc.get_sparse_core_info()
    n_workers = info.num_cores * info.num_subcores

    assert b % (n_workers * _G) == 0
    return _sc_gather(input.astype(jnp.int32), embedding_matrix, n_workers)


# speedup vs baseline: 3.9807x; 1.0124x over previous
import functools

import jax
import jax.numpy as jnp
from jax import lax
from jax.experimental import pallas as pl
from jax.experimental.pallas import tpu as pltpu
from jax.experimental.pallas import tpu_sc as plsc

_G = 32


@functools.partial(jax.jit, static_argnums=(2,))
def _sc_gather(idx, table, n_workers):
    b, h = idx.shape
    _, d = table.shape
    b_per_w = b // n_workers
    groups = b_per_w // _G
    assert groups % 2 == 0 and groups >= 4

    mesh = plsc.VectorSubcoreMesh(core_axis_name="c", subcore_axis_name="s")

    @functools.partial(
        pl.kernel,
        out_type=jax.ShapeDtypeStruct((b, h, d), jnp.float32),
        mesh=mesh,
        scratch_types=[
            pltpu.VMEM((_G, h), jnp.int32),
            pltpu.VMEM((_G, h), jnp.int32),
            pltpu.VMEM((_G, h, d), jnp.float32),
            pltpu.VMEM((_G, h, d), jnp.float32),
            pltpu.SemaphoreType.DMA,
            pltpu.SemaphoreType.DMA,
            pltpu.SemaphoreType.DMA,
            pltpu.SemaphoreType.DMA,
        ],
        compiler_params=pltpu.CompilerParams(use_tc_tiling_on_sc=False),
    )
    def k(idx_hbm, table_hbm, out_hbm, idx0, idx1, rows0, rows1,
          gsem0, gsem1, ssem0, ssem1):
        idxv = (idx0, idx1)
        rows = (rows0, rows1)
        gsem = (gsem0, gsem1)
        ssem = (ssem0, ssem1)

        nc = lax.axis_size("c")
        wid = lax.axis_index("s") * nc + lax.axis_index("c")
        b0 = wid * b_per_w

        def fire_gathers(g, buf):
            pltpu.sync_copy(idx_hbm.at[pl.ds(b0 + g * _G, _G)], idxv[buf])
            for j in range(_G):
                pltpu.async_copy(
                    table_hbm.at[idxv[buf].at[j]],
                    rows[buf].at[j],
                    gsem[buf],
                )

        def drain_gathers(buf):
            for j in range(_G):
                pltpu.make_async_copy(
                    table_hbm.at[idxv[buf].at[j]],
                    rows[buf].at[j],
                    gsem[buf],
                ).wait()

        def fire_store(g, buf):
            pltpu.async_copy(
                rows[buf], out_hbm.at[pl.ds(b0 + g * _G, _G)], ssem[buf])

        def drain_store(g, buf):
            pltpu.make_async_copy(
                rows[buf], out_hbm.at[pl.ds(b0 + g * _G, _G)], ssem[buf]
            ).wait()

        fire_gathers(0, 0)
        fire_gathers(1, 1)
        drain_gathers(0)
        fire_store(0, 0)

        def body(j, carry):
            g = 2 * j + 1
            drain_store(g - 1, 0)
            fire_gathers(g + 1, 0)
            drain_gathers(1)
            fire_store(g, 1)

            drain_store(g, 1)
            fire_gathers(g + 2, 1)
            drain_gathers(0)
            fire_store(g + 1, 0)
            return carry

        lax.fori_loop(0, (groups - 2) // 2, body, 0)

        drain_gathers(1)
        fire_store(groups - 1, 1)
        drain_store(groups - 2, 0)
        drain_store(groups - 1, 1)

    return k(idx, table)


def kernel(input, embedding_matrix):
    b, h = input.shape

    info = plsc.get_sparse_core_info()
    n_workers = info.num_cores * info.num_subcores

    assert b % (n_workers * _G) == 0
    return _sc_gather(input.astype(jnp.int32), embedding_matrix, n_workers)
